# baseline (speedup 1.0000x reference)
import jax
import jax.numpy as jnp
from jax.experimental import pallas as pl
from jax.experimental.pallas import tpu as pltpu

_B = 4
_N = 4096
_ROW_TILE = 2048
_MXU_COLS = 3328
_VPU_COLS = _N - _MXU_COLS


def _chamfer_body(x_ref, y_ref, xt_ref, out_ref):
    total = jnp.float32(0.0)
    bf = jnp.bfloat16
    f32 = jnp.float32
    for b in range(_B):
        x = x_ref[b]
        y = y_ref[b]
        xt = xt_ref[b]

        nx = x[0:1, :] ** 2 + x[1:2, :] ** 2 + x[2:3, :] ** 2
        nxhi = nx.astype(bf)
        nxlo = (nx - nxhi.astype(f32)).astype(bf)
        a_aug = jnp.concatenate(
            [x.astype(bf), nxhi, nxlo, jnp.ones((2, _N), dtype=bf)],
            axis=0,
        )

        ny = y[0:1, :] ** 2 + y[1:2, :] ** 2 + y[2:3, :] ** 2
        nyhi = ny.astype(bf)
        nylo = (ny - nyhi.astype(f32)).astype(bf)
        b_aug = jnp.concatenate(
            [
                jnp.bfloat16(-2.0) * y.astype(bf),
                jnp.ones((2, _N), dtype=bf),
                nyhi,
                nylo,
            ],
            axis=0,
        )

        ys = y[:, _MXU_COLS:]
        ysb = ys.astype(bf).astype(f32)
        nys = ny[:, _MXU_COLS:]

        row_sum = jnp.float32(0.0)
        cm_m = jnp.full((1, _MXU_COLS), jnp.inf, dtype=f32)
        cm_v = jnp.full((1, _VPU_COLS), jnp.inf, dtype=f32)
        for t in range(_N // _ROW_TILE):
            lo = t * _ROW_TILE
            dist_m = jax.lax.dot_general(
                a_aug[:, lo:lo + _ROW_TILE], b_aug[:, :_MXU_COLS],
                dimension_numbers=(((0,), (0,)), ((), ())),
                preferred_element_type=f32,
            )
            xs = xt[lo:lo + _ROW_TILE, :]
            xsb = xs.astype(bf).astype(f32)
            nxs = jnp.sum(xs * xs, axis=1, keepdims=True)
            prod = (
                xsb[:, 0:1] * ysb[0:1, :]
                + xsb[:, 1:2] * ysb[1:2, :]
                + xsb[:, 2:3] * ysb[2:3, :]
            )
            dist_v = (-2.0 * prod + nxs) + nys

            row_min = jnp.minimum(
                jnp.min(dist_m, axis=1), jnp.min(dist_v, axis=1)
            )
            row_sum = row_sum + jnp.sum(row_min)
            cm_m = jnp.minimum(cm_m, jnp.min(dist_m, axis=0, keepdims=True))
            cm_v = jnp.minimum(cm_v, jnp.min(dist_v, axis=0, keepdims=True))
        total = total + row_sum + jnp.sum(cm_m) + jnp.sum(cm_v)

    out_ref[0, 0] = total * jnp.float32(1.0 / (2.0 * _B * _N))


def kernel(in_pc, target_pc):
    in_t = jnp.transpose(in_pc, (0, 2, 1))
    total = pl.pallas_call(
        _chamfer_body,
        out_specs=pl.BlockSpec(memory_space=pltpu.SMEM),
        out_shape=jax.ShapeDtypeStruct((1, 1), jnp.float32),
    )(in_pc, target_pc, in_t)
    return total[0, 0]


# speedup vs baseline: 1.3333x; 1.3333x over previous
import jax
import jax.numpy as jnp
from jax.experimental import pallas as pl
from jax.experimental.pallas import tpu as pltpu

_B = 4
_N = 4096
_ROW_TILE = 2048


def _chamfer_body(x_ref, y_ref, out_ref):
    total = jnp.float32(0.0)
    for b in range(_B):
        x = x_ref[b]
        y = y_ref[b]

        nx = x[0:1, :] ** 2 + x[1:2, :] ** 2 + x[2:3, :] ** 2
        nxhi = nx.astype(jnp.bfloat16)
        nxlo = (nx - nxhi.astype(jnp.float32)).astype(jnp.bfloat16)
        a_aug = jnp.concatenate(
            [
                x.astype(jnp.bfloat16),
                nxhi,
                nxlo,
                jnp.ones((2, _N), dtype=jnp.bfloat16),
            ],
            axis=0,
        )

        ny = y[0:1, :] ** 2 + y[1:2, :] ** 2 + y[2:3, :] ** 2
        nyhi = ny.astype(jnp.bfloat16)
        nylo = (ny - nyhi.astype(jnp.float32)).astype(jnp.bfloat16)
        b_aug = jnp.concatenate(
            [
                jnp.bfloat16(-2.0) * y.astype(jnp.bfloat16),
                jnp.ones((2, _N), dtype=jnp.bfloat16),
                nyhi,
                nylo,
            ],
            axis=0,
        )

        row_sum = jnp.float32(0.0)
        col_min = jnp.full((1, _N), jnp.inf, dtype=jnp.float32)
        for t in range(_N // _ROW_TILE):
            lo = t * _ROW_TILE
            dist = jax.lax.dot_general(
                a_aug[:, lo:lo + _ROW_TILE], b_aug,
                dimension_numbers=(((0,), (0,)), ((), ())),
                preferred_element_type=jnp.float32,
            )
            row_min = jnp.min(dist, axis=1)
            row_sum = row_sum + jnp.sum(row_min)
            col_min = jnp.minimum(
                col_min, jnp.min(dist, axis=0, keepdims=True)
            )
        total = total + row_sum + jnp.sum(col_min)

    out_ref[0, 0] = total * jnp.float32(1.0 / (2.0 * _B * _N))


def kernel(in_pc, target_pc):
    total = pl.pallas_call(
        _chamfer_body,
        out_specs=pl.BlockSpec(memory_space=pltpu.SMEM),
        out_shape=jax.ShapeDtypeStruct((1, 1), jnp.float32),
    )(in_pc, target_pc)
    return total[0, 0]
